# baseline (speedup 1.0000x reference)
import functools

import jax
import jax.numpy as jnp
from jax import lax
from jax.experimental import pallas as pl
from jax.experimental.pallas import tpu as pltpu
from jax.experimental.pallas import tpu_sc as plsc

D_CONST = 1024
LANES = 16


def _make_sc_kernel(B, S, D):
    info = plsc.get_sparse_core_info()
    NC, NS = info.num_cores, info.num_subcores
    NW = NC * NS
    s_per_w = S // NW
    CH = 16
    n_ch = s_per_w // CH
    words = CH * D
    n_vec = words // LANES
    UNROLL = 8

    mesh = plsc.VectorSubcoreMesh(core_axis_name="c", subcore_axis_name="s")

    @functools.partial(
        pl.kernel,
        mesh=mesh,
        out_type=jax.ShapeDtypeStruct((B * S * D,), jnp.float32),
        scratch_types=[
            pltpu.VMEM((words,), jnp.float32),
            pltpu.VMEM((words,), jnp.float32),
        ],
    )
    def k(xf, pe, out, x_buf, pe_buf):
        wid = lax.axis_index("s") * NC + lax.axis_index("c")
        s_base = wid * s_per_w

        def add_block(i, _):
            base = i * (LANES * UNROLL)
            for u in range(UNROLL):
                off = base + u * LANES
                vec = pe_buf[pl.ds(off, LANES)]
                plsc.addupdate(x_buf.at[pl.ds(off, LANES)], vec)
            return _

        for c in range(n_ch):
            s0 = s_base + c * CH
            pltpu.sync_copy(pe.at[pl.ds(s0 * D, words)], pe_buf)
            for b in range(B):
                row0 = b * S + s0
                pltpu.sync_copy(xf.at[pl.ds(row0 * D, words)], x_buf)
                lax.fori_loop(0, n_vec // UNROLL, add_block, 0)
                pltpu.sync_copy(x_buf, out.at[pl.ds(row0 * D, words)])

    return k


def kernel(x, pe_weight):
    B, S, D = x.shape
    pe = pe_weight[:S].reshape(S * D)
    xf = x.reshape(B * S * D)
    out = _make_sc_kernel(B, S, D)(xf, pe)
    return out.reshape(B, S, D)


# speedup vs baseline: 1.2301x; 1.2301x over previous
import functools

import jax
import jax.numpy as jnp
from jax import lax
from jax.experimental import pallas as pl
from jax.experimental.pallas import tpu as pltpu
from jax.experimental.pallas import tpu_sc as plsc

LANES = 16
NBUF = 3


def _make_sc_kernel(B, S, D):
    info = plsc.get_sparse_core_info()
    NC, NS = info.num_cores, info.num_subcores
    NW = NC * NS
    s_per_w = S // NW
    CH = 8
    n_ch = s_per_w // CH
    chw = CH * D
    UNROLL = 8

    mesh = plsc.VectorSubcoreMesh(core_axis_name="c", subcore_axis_name="s")

    scratch = (
        [pltpu.VMEM((B * chw,), jnp.float32) for _ in range(NBUF)]
        + [pltpu.VMEM((chw,), jnp.float32) for _ in range(NBUF)]
        + [pltpu.SemaphoreType.DMA for _ in range(2 * NBUF)]
    )

    @functools.partial(
        pl.kernel,
        mesh=mesh,
        out_type=jax.ShapeDtypeStruct((B * S * D,), jnp.float32),
        scratch_types=scratch,
    )
    def k(xf, pe, out, xb0, xb1, xb2, pb0, pb1, pb2,
          ls0, ls1, ls2, ss0, ss1, ss2):
        xbs = (xb0, xb1, xb2)
        pbs = (pb0, pb1, pb2)
        lss = (ls0, ls1, ls2)
        sss = (ss0, ss1, ss2)

        wid = lax.axis_index("s") * NC + lax.axis_index("c")
        s_base = wid * s_per_w

        def start_loads(c):
            p = c % NBUF
            s0 = s_base + c * CH
            hs = [pltpu.async_copy(pe.at[pl.ds(s0 * D, chw)], pbs[p], lss[p])]
            for b in range(B):
                off = (b * S + s0) * D
                hs.append(
                    pltpu.async_copy(
                        xf.at[pl.ds(off, chw)],
                        xbs[p].at[pl.ds(b * chw, chw)],
                        lss[p],
                    )
                )
            return hs

        def start_stores(c):
            p = c % NBUF
            s0 = s_base + c * CH
            hs = []
            for b in range(B):
                off = (b * S + s0) * D
                hs.append(
                    pltpu.async_copy(
                        xbs[p].at[pl.ds(b * chw, chw)],
                        out.at[pl.ds(off, chw)],
                        sss[p],
                    )
                )
            return hs

        def compute(c):
            p = c % NBUF
            xb, pb = xbs[p], pbs[p]

            def body(i, carry):
                base = i * (LANES * UNROLL)
                for u in range(UNROLL):
                    off = base + u * LANES
                    vec = pb[pl.ds(off, LANES)]
                    for b in range(B):
                        plsc.addupdate(xb.at[pl.ds(b * chw + off, LANES)], vec)
                return carry

            lax.fori_loop(0, chw // (LANES * UNROLL), body, 0)

        loads = {c: start_loads(c) for c in range(min(NBUF, n_ch))}
        stores = {}
        for c in range(n_ch):
            if c >= NBUF - 1:
                for h in stores.pop(c - (NBUF - 1)):
                    h.wait()
                if c + 1 < n_ch:
                    loads[c + 1] = start_loads(c + 1)
            for h in loads.pop(c):
                h.wait()
            compute(c)
            stores[c] = start_stores(c)
        for hs in stores.values():
            for h in hs:
                h.wait()

    return k


def kernel(x, pe_weight):
    B, S, D = x.shape
    pe = pe_weight[:S].reshape(S * D)
    xf = x.reshape(B * S * D)
    out = _make_sc_kernel(B, S, D)(xf, pe)
    return out.reshape(B, S, D)


# speedup vs baseline: 1.2814x; 1.0417x over previous
import functools

import jax
import jax.numpy as jnp
from jax import lax
from jax.experimental import pallas as pl
from jax.experimental.pallas import tpu as pltpu
from jax.experimental.pallas import tpu_sc as plsc

LANES = 16
NBUF = 3


def _make_sc_kernel(B, S, D):
    info = plsc.get_sparse_core_info()
    NC, NS = info.num_cores, info.num_subcores
    NW = NC * NS
    s_per_w = S // NW
    CH = 8
    n_ch = s_per_w // CH
    chw = CH * D
    UNROLL = 8

    mesh = plsc.VectorSubcoreMesh(core_axis_name="c", subcore_axis_name="s")

    scratch = (
        [pltpu.VMEM((B, chw), jnp.float32) for _ in range(NBUF)]
        + [pltpu.VMEM((chw,), jnp.float32) for _ in range(NBUF)]
        + [pltpu.SemaphoreType.DMA for _ in range(2 * NBUF)]
    )

    @functools.partial(
        pl.kernel,
        mesh=mesh,
        out_type=jax.ShapeDtypeStruct((B, S * D), jnp.float32),
        scratch_types=scratch,
    )
    def k(xf, pe, out, xb0, xb1, xb2, pb0, pb1, pb2,
          ls0, ls1, ls2, ss0, ss1, ss2):
        xbs = (xb0, xb1, xb2)
        pbs = (pb0, pb1, pb2)
        lss = (ls0, ls1, ls2)
        sss = (ss0, ss1, ss2)

        wid = lax.axis_index("s") * NC + lax.axis_index("c")
        s_base = wid * s_per_w

        def start_loads(c):
            p = c % NBUF
            s0 = s_base + c * CH
            return [
                pltpu.async_copy(pe.at[pl.ds(s0 * D, chw)], pbs[p], lss[p]),
                pltpu.async_copy(
                    xf.at[:, pl.ds(s0 * D, chw)], xbs[p], lss[p]
                ),
            ]

        def start_stores(c):
            p = c % NBUF
            s0 = s_base + c * CH
            return [
                pltpu.async_copy(
                    xbs[p], out.at[:, pl.ds(s0 * D, chw)], sss[p]
                ),
            ]

        def compute(c):
            p = c % NBUF
            xb, pb = xbs[p], pbs[p]

            def body(i, carry):
                base = i * (LANES * UNROLL)
                for u in range(UNROLL):
                    off = base + u * LANES
                    vec = pb[pl.ds(off, LANES)]
                    for b in range(B):
                        plsc.addupdate(xb.at[b, pl.ds(off, LANES)], vec)
                return carry

            lax.fori_loop(0, chw // (LANES * UNROLL), body, 0)

        loads = {c: start_loads(c) for c in range(min(NBUF, n_ch))}
        stores = {}
        for c in range(n_ch):
            if c >= NBUF - 1:
                for h in stores.pop(c - (NBUF - 1)):
                    h.wait()
                if c + 1 < n_ch:
                    loads[c + 1] = start_loads(c + 1)
            for h in loads.pop(c):
                h.wait()
            compute(c)
            stores[c] = start_stores(c)
        for hs in stores.values():
            for h in hs:
                h.wait()

    return k


def kernel(x, pe_weight):
    B, S, D = x.shape
    pe = pe_weight[:S].reshape(S * D)
    xf = x.reshape(B, S * D)
    out = _make_sc_kernel(B, S, D)(xf, pe)
    return out.reshape(B, S, D)


# speedup vs baseline: 2.6265x; 2.0497x over previous
import functools

import jax
import jax.numpy as jnp
from jax import lax
from jax.experimental import pallas as pl
from jax.experimental.pallas import tpu as pltpu
from jax.experimental.pallas import tpu_sc as plsc

LANES = 16
NBUF = 3


def _make_sc_kernel(B, S, D):
    info = plsc.get_sparse_core_info()
    NC, NS = info.num_cores, info.num_subcores
    NW = NC * NS
    s_per_w = S // NW
    CH = 8
    n_ch = s_per_w // CH
    n_col = D // LANES

    mesh = plsc.VectorSubcoreMesh(core_axis_name="c", subcore_axis_name="s")

    scratch = (
        [pltpu.VMEM((B, CH, D), jnp.float32) for _ in range(NBUF)]
        + [pltpu.VMEM((CH, D), jnp.float32) for _ in range(NBUF)]
        + [pltpu.SemaphoreType.DMA for _ in range(2 * NBUF)]
    )

    @functools.partial(
        pl.kernel,
        mesh=mesh,
        out_type=jax.ShapeDtypeStruct((B, S, D), jnp.float32),
        scratch_types=scratch,
        compiler_params=pltpu.CompilerParams(use_tc_tiling_on_sc=True),
    )
    def k(xf, pe, out, xb0, xb1, xb2, pb0, pb1, pb2,
          ls0, ls1, ls2, ss0, ss1, ss2):
        xbs = (xb0, xb1, xb2)
        pbs = (pb0, pb1, pb2)
        lss = (ls0, ls1, ls2)
        sss = (ss0, ss1, ss2)

        wid = lax.axis_index("s") * NC + lax.axis_index("c")
        s_base = wid * s_per_w

        def start_loads(c):
            p = c % NBUF
            s0 = s_base + c * CH
            return [
                pltpu.async_copy(pe.at[pl.ds(s0, CH), :], pbs[p], lss[p]),
                pltpu.async_copy(xf.at[:, pl.ds(s0, CH), :], xbs[p], lss[p]),
            ]

        def start_stores(c):
            p = c % NBUF
            s0 = s_base + c * CH
            return [
                pltpu.async_copy(xbs[p], out.at[:, pl.ds(s0, CH), :], sss[p]),
            ]

        def compute(c):
            p = c % NBUF
            xb, pb = xbs[p], pbs[p]

            def body(r, carry):
                for g in range(n_col):
                    col = g * LANES
                    vec = pb[r, pl.ds(col, LANES)]
                    for b in range(B):
                        plsc.addupdate(xb.at[b, r, pl.ds(col, LANES)], vec)
                return carry

            lax.fori_loop(0, CH, body, 0)

        loads = {c: start_loads(c) for c in range(min(NBUF, n_ch))}
        stores = {}
        for c in range(n_ch):
            if c >= NBUF - 1:
                for h in stores.pop(c - (NBUF - 1)):
                    h.wait()
                if c + 1 < n_ch:
                    loads[c + 1] = start_loads(c + 1)
            for h in loads.pop(c):
                h.wait()
            compute(c)
            stores[c] = start_stores(c)
        for hs in stores.values():
            for h in hs:
                h.wait()

    return k


def kernel(x, pe_weight):
    B, S, D = x.shape
    return _make_sc_kernel(B, S, D)(x, pe_weight[:S])
